# Initial kernel scaffold; baseline (speedup 1.0000x reference)
import functools
import jax
import jax.numpy as jnp
from jax import lax
from jax.experimental import pallas as pl
from jax.experimental.pallas import tpu as pltpu
from jax.experimental.pallas import tpu_sc as plsc

VOCAB = 100000
NADD = 1024
D = 64
BATCH = 4096
HIST = 200
B = BATCH * HIST
NC, NS = 2, 16
NW = NC * NS
BPW = B // NW
CH = 512
NCHUNK = BPW // CH
GW = 128
NSUB = CH // GW

_mesh = plsc.VectorSubcoreMesh(core_axis_name="c", subcore_axis_name="s")


@functools.partial(
    pl.kernel,
    mesh=_mesh,
    out_type=jax.ShapeDtypeStruct((B, D), jnp.float32),
    scratch_types=[
        pltpu.VMEM((NSUB, GW), jnp.int32),
        pltpu.VMEM((CH, D), jnp.float32),
        pltpu.SemaphoreType.DMA,
        pltpu.SemaphoreType.DMA,
    ],
)
def _gather_kernel(table_hbm, idx_hbm, out_hbm, idx_v, rows_v, gsem, osem):
    wid = lax.axis_index("s") * NC + lax.axis_index("c")
    base = wid * BPW

    def chunk_body(c, _):
        cbase = base + c * CH
        pltpu.sync_copy(idx_hbm.at[pl.ds(cbase // GW, NSUB)], idx_v)
        for j in range(NSUB):
            pltpu.async_copy(
                table_hbm.at[idx_v.at[j]],
                rows_v.at[pl.ds(j * GW, GW)],
                gsem,
            )
        for j in range(NSUB):
            pltpu.make_async_copy(
                table_hbm.at[idx_v.at[j]],
                rows_v.at[pl.ds(j * GW, GW)],
                gsem,
            ).wait()
        pltpu.async_copy(rows_v, out_hbm.at[pl.ds(cbase, CH)], osem).wait()
        return ()

    lax.fori_loop(0, NCHUNK, chunk_body, ())


@jax.jit
def _impl(embed_frozen, weights_train, idx):
    table = jnp.concatenate((embed_frozen, weights_train), axis=0)
    idx2 = idx.reshape(B // GW, GW).astype(jnp.int32)
    out = _gather_kernel(table, idx2)
    return out.reshape(BATCH, HIST, D)


def kernel(embed_frozen, weights_train, idx):
    return _impl(embed_frozen, weights_train, idx)


# speedup vs baseline: 4.9881x; 4.9881x over previous
import functools
import jax
import jax.numpy as jnp
from jax import lax
from jax.experimental import pallas as pl
from jax.experimental.pallas import tpu as pltpu
from jax.experimental.pallas import tpu_sc as plsc

VOCAB = 100000
NADD = 1024
D = 64
BATCH = 4096
HIST = 200
B = BATCH * HIST
NC, NS = 2, 16
NW = NC * NS
BPW = B // NW
CH = 512
NCHUNK = BPW // CH
GW = 128
NSUB = CH // GW

_mesh = plsc.VectorSubcoreMesh(core_axis_name="c", subcore_axis_name="s")


@functools.partial(
    pl.kernel,
    mesh=_mesh,
    out_type=jax.ShapeDtypeStruct((B, D), jnp.float32),
    scratch_types=[
        pltpu.VMEM((CH,), jnp.int32),
        pltpu.VMEM((CH, D), jnp.float32),
        pltpu.SemaphoreType.DMA,
        pltpu.SemaphoreType.DMA,
    ],
    compiler_params=pltpu.CompilerParams(use_tc_tiling_on_sc=False),
)
def _gather_kernel(table_hbm, idx_hbm, out_hbm, idx_v, rows_v, gsem, osem):
    wid = lax.axis_index("s") * NC + lax.axis_index("c")
    base = wid * BPW

    def chunk_body(c, _):
        cbase = base + c * CH
        pltpu.sync_copy(idx_hbm.at[pl.ds(cbase, CH)], idx_v)
        for j in range(NSUB):
            pltpu.async_copy(
                table_hbm.at[idx_v.at[pl.ds(j * GW, GW)]],
                rows_v.at[pl.ds(j * GW, GW)],
                gsem,
            )
        for j in range(NSUB):
            pltpu.make_async_copy(
                table_hbm.at[idx_v.at[pl.ds(j * GW, GW)]],
                rows_v.at[pl.ds(j * GW, GW)],
                gsem,
            ).wait()
        pltpu.async_copy(rows_v, out_hbm.at[pl.ds(cbase, CH)], osem).wait()
        return ()

    lax.fori_loop(0, NCHUNK, chunk_body, ())


@jax.jit
def _impl(embed_frozen, weights_train, idx):
    table = jnp.concatenate((embed_frozen, weights_train), axis=0)
    idx2 = idx.reshape(B).astype(jnp.int32)
    out = _gather_kernel(table, idx2)
    return out.reshape(BATCH, HIST, D)


def kernel(embed_frozen, weights_train, idx):
    return _impl(embed_frozen, weights_train, idx)
